# baseline (speedup 1.0000x reference)
import functools

import jax
import jax.numpy as jnp
from jax import lax
from jax.experimental import pallas as pl
from jax.experimental.pallas import tpu as pltpu
from jax.experimental.pallas import tpu_sc as plsc

N_NODES = 10000
N_TILES = 16
WIN = 32
NBUF = 8
PAD_ROWS = 8


def _sc_agg(h2, srcb, dstb, fh, kw, seed_stride):
    n = N_NODES
    rpt = 632
    rlast = n - 15 * rpt
    mesh = plsc.VectorSubcoreMesh(core_axis_name="c", subcore_axis_name="s")

    kwh = 40
    nst = kw // kwh

    @functools.partial(
        pl.kernel,
        mesh=mesh,
        out_type=jax.ShapeDtypeStruct((2, n, fh), jnp.float32),
        scratch_types=[
            pltpu.VMEM((kwh, WIN), jnp.int32),
            pltpu.VMEM((kwh, WIN), jnp.int32),
            pltpu.VMEM((NBUF, WIN, fh), jnp.float32),
            pltpu.VMEM_SHARED((n + PAD_ROWS, fh), jnp.float32),
            pltpu.SemaphoreType.DMA,
            pltpu.SemaphoreType.DMA,
        ],
    )
    def agg(h_hbm, src_hbm, dst_hbm, out_hbm, src_v, dst_v, rows_v, acc, sem,
            sem_s):
        c = lax.axis_index("c")
        s = lax.axis_index("s")
        r0 = s * rpt

        @pl.when(s < 15)
        def _():
            pltpu.sync_copy(h_hbm.at[pl.ds(c * seed_stride + r0, rpt)],
                            acc.at[pl.ds(r0, rpt)])

        @pl.when(s == 15)
        def _():
            pltpu.sync_copy(h_hbm.at[pl.ds(c * seed_stride + 15 * rpt, rlast)],
                            acc.at[pl.ds(15 * rpt, rlast)])

        plsc.subcore_barrier()

        def winN(i, carry):
            j0 = i * NBUF
            for b in range(NBUF):
                j = j0 + b
                pltpu.make_async_copy(
                    h_hbm.at[src_v.at[j]], rows_v.at[b], sem).wait()
                pltpu.async_copy(rows_v.at[b], acc.at[dst_v.at[j]], sem_s,
                                 add=True)
                nxt = j + NBUF - 1
                bn_ = (b + NBUF - 1) % NBUF

                @pl.when((nxt < kwh) & (j >= 1))
                def _():
                    pltpu.make_async_copy(
                        rows_v.at[bn_], acc.at[dst_v.at[j]], sem_s).wait()

                @pl.when(nxt < kwh)
                def _():
                    pltpu.async_copy(
                        h_hbm.at[src_v.at[nxt]], rows_v.at[bn_], sem)
            return carry

        for stage in range(nst):
            base = s * kw + stage * kwh
            pltpu.sync_copy(src_hbm.at[c, pl.ds(base, kwh)], src_v)
            pltpu.sync_copy(dst_hbm.at[c, pl.ds(base, kwh)], dst_v)
            for k in range(NBUF - 1):
                pltpu.async_copy(h_hbm.at[src_v.at[k]], rows_v.at[k], sem)
            lax.fori_loop(0, kwh // NBUF, winN, 0)
            for k in range(NBUF):
                pltpu.make_async_copy(
                    rows_v.at[k], acc.at[dst_v.at[0]], sem_s).wait()

        plsc.subcore_barrier()

        @pl.when(s < 15)
        def _():
            pltpu.sync_copy(acc.at[pl.ds(r0, rpt)],
                            out_hbm.at[c, pl.ds(r0, rpt)])

        @pl.when(s == 15)
        def _():
            pltpu.sync_copy(acc.at[pl.ds(15 * rpt, rlast)],
                            out_hbm.at[c, pl.ds(15 * rpt, rlast)])

    return agg(h2, srcb, dstb)


def _tc_mlp1(p01, x, batch3, w1, b1, w2, b2):
    n, bn = N_NODES, 1000

    def body(agg_r, x_r, bat_r, w1_r, b1_r, w2_r, b2_r, h1_r, p1_r):
        a = agg_r[0] + agg_r[1] - x_r[...]
        t = jnp.dot(a, w1_r[...], preferred_element_type=jnp.float32) + b1_r[...]
        t = jnp.maximum(t, 0.0)
        h1 = jnp.dot(t, w2_r[...], preferred_element_type=jnp.float32) + b2_r[...]
        h1 = jnp.maximum(h1, 0.0)
        h1_r[0] = h1[:, :128]
        h1_r[1] = h1[:, 128:]
        gids = lax.broadcasted_iota(jnp.int32, (64, bn), 0)
        oh = (bat_r[0] == gids).astype(jnp.float32)
        part = jnp.dot(oh, h1, preferred_element_type=jnp.float32)

        @pl.when(pl.program_id(0) == 0)
        def _():
            p1_r[...] = jnp.zeros_like(p1_r)

        p1_r[...] += part

    return pl.pallas_call(
        body,
        grid=(n // bn,),
        in_specs=[
            pl.BlockSpec((2, bn, 128), lambda i: (0, i, 0)),
            pl.BlockSpec((bn, 128), lambda i: (i, 0)),
            pl.BlockSpec((1, 1, bn), lambda i: (i, 0, 0)),
            pl.BlockSpec((128, 256), lambda i: (0, 0)),
            pl.BlockSpec((1, 256), lambda i: (0, 0)),
            pl.BlockSpec((256, 256), lambda i: (0, 0)),
            pl.BlockSpec((1, 256), lambda i: (0, 0)),
        ],
        out_specs=[
            pl.BlockSpec((2, bn, 128), lambda i: (0, i, 0)),
            pl.BlockSpec((64, 256), lambda i: (0, 0)),
        ],
        out_shape=[
            jax.ShapeDtypeStruct((2, n, 128), jnp.float32),
            jax.ShapeDtypeStruct((64, 256), jnp.float32),
        ],
    )(p01, x, batch3, w1, b1, w2, b2)


def _tc_mlp2(agg2, batch3, w3, b3, w4, b4):
    n, bn = N_NODES, 1000

    def body(agg_r, bat_r, w3_r, b3_r, w4_r, b4_r, p2_r):
        a = jnp.concatenate([agg_r[0], agg_r[1]], axis=1)
        t = jnp.dot(a, w3_r[...], preferred_element_type=jnp.float32) + b3_r[...]
        t = jnp.maximum(t, 0.0)
        h2 = jnp.dot(t, w4_r[...], preferred_element_type=jnp.float32) + b4_r[...]
        h2 = jnp.maximum(h2, 0.0)
        gids = lax.broadcasted_iota(jnp.int32, (64, bn), 0)
        oh = (bat_r[0] == gids).astype(jnp.float32)
        part = jnp.dot(oh, h2, preferred_element_type=jnp.float32)

        @pl.when(pl.program_id(0) == 0)
        def _():
            p2_r[...] = jnp.zeros_like(p2_r)

        p2_r[...] += part

    return pl.pallas_call(
        body,
        grid=(n // bn,),
        in_specs=[
            pl.BlockSpec((2, bn, 128), lambda i: (0, i, 0)),
            pl.BlockSpec((1, 1, bn), lambda i: (i, 0, 0)),
            pl.BlockSpec((256, 256), lambda i: (0, 0)),
            pl.BlockSpec((1, 256), lambda i: (0, 0)),
            pl.BlockSpec((256, 256), lambda i: (0, 0)),
            pl.BlockSpec((1, 256), lambda i: (0, 0)),
        ],
        out_specs=pl.BlockSpec((64, 256), lambda i: (0, 0)),
        out_shape=jax.ShapeDtypeStruct((64, 256), jnp.float32),
    )(agg2, batch3, w3, b3, w4, b4)


def _tc_head(p1, p2, wl1, bl1, wl2, bl2):
    def body(p1_r, p2_r, w1_r, b1_r, w2_r, b2_r, o_r):
        z = (jnp.dot(p1_r[...], w1_r[0], preferred_element_type=jnp.float32)
             + jnp.dot(p2_r[...], w1_r[1], preferred_element_type=jnp.float32)
             + b1_r[...])
        z = jnp.maximum(z, 0.0)
        o_r[...] = jnp.dot(z, w2_r[...], preferred_element_type=jnp.float32) + b2_r[...]

    return pl.pallas_call(
        body,
        out_shape=jax.ShapeDtypeStruct((64, 10), jnp.float32),
    )(p1, p2, wl1, bl1, wl2, bl2)


def kernel(x, edge_index, batch, W1, b1, g1, be1, W2, b2, W3, b3, g2, be2,
           W4, b4, Wl1, bl1, Wl2, bl2):
    n = x.shape[0]
    e = edge_index.shape[1]
    src = edge_index[0]
    dst = edge_index[1]

    eh = e // 2
    kw1 = -(-(-(-eh // (N_TILES * WIN))) // 8) * 8
    npad1 = N_TILES * kw1 * WIN - eh
    pad1 = jnp.arange(npad1, dtype=jnp.int32) % PAD_ROWS
    srcb1 = jnp.stack([
        jnp.concatenate([src[:eh], pad1]),
        jnp.concatenate([src[eh:], pad1]),
    ]).reshape(2, N_TILES * kw1, WIN)
    dstb1 = jnp.stack([
        jnp.concatenate([dst[:eh], n + pad1]),
        jnp.concatenate([dst[eh:], n + pad1]),
    ]).reshape(2, N_TILES * kw1, WIN)

    kw2 = -(-(-(-e // (N_TILES * WIN))) // 8) * 8
    npad2 = N_TILES * kw2 * WIN - e
    pad2 = jnp.arange(npad2, dtype=jnp.int32) % PAD_ROWS
    srcp2 = jnp.concatenate([src, pad2])
    dstp2 = jnp.concatenate([dst, n + pad2])
    srcb2 = jnp.stack([srcp2, srcp2 + n]).reshape(2, N_TILES * kw2, WIN)
    dstb2 = jnp.stack([dstp2, dstp2]).reshape(2, N_TILES * kw2, WIN)

    sinv = 1.0 / jnp.sqrt(1.0 + 1e-5)
    w1s = g1 * sinv
    W1f = W1 * w1s[None, :]
    b1f = b1 * w1s + be1
    w4s = g2 * sinv
    W4f = W4 * w4s[:, None]
    b4f = b4 + be2 @ W4

    batch3 = batch.reshape(10, 1, 1000)

    agg1 = _sc_agg(x, srcb1, dstb1, 128, kw1, 0)
    h1_both, p1 = _tc_mlp1(agg1, x, batch3, W1f, b1f.reshape(1, -1), W2,
                           b2.reshape(1, -1))

    agg2 = _sc_agg(h1_both.reshape(2 * n, 128), srcb2, dstb2, 128, kw2, n)
    p2 = _tc_mlp2(agg2, batch3, W3, b3.reshape(1, -1), W4f,
                  b4f.reshape(1, -1))

    return _tc_head(p1, p2, Wl1.reshape(2, 256, 128), bl1.reshape(1, -1),
                    Wl2, bl2.reshape(1, -1))


# speedup vs baseline: 1.0177x; 1.0177x over previous
import functools

import jax
import jax.numpy as jnp
from jax import lax
from jax.experimental import pallas as pl
from jax.experimental.pallas import tpu as pltpu
from jax.experimental.pallas import tpu_sc as plsc

N_NODES = 10000
N_TILES = 16
WIN = 64
NBUF = 4
PAD_ROWS = 8


def _sc_agg(h2, srcb, dstb, fh, kw, seed_stride):
    n = N_NODES
    rpt = 632
    rlast = n - 15 * rpt
    mesh = plsc.VectorSubcoreMesh(core_axis_name="c", subcore_axis_name="s")

    kwh = 40
    nst = kw // kwh

    @functools.partial(
        pl.kernel,
        mesh=mesh,
        out_type=jax.ShapeDtypeStruct((2, n, fh), jnp.float32),
        scratch_types=[
            pltpu.VMEM((kwh, WIN), jnp.int32),
            pltpu.VMEM((kwh, WIN), jnp.int32),
            pltpu.VMEM((NBUF, WIN, fh), jnp.float32),
            pltpu.VMEM_SHARED((n + PAD_ROWS, fh), jnp.float32),
            pltpu.SemaphoreType.DMA,
            pltpu.SemaphoreType.DMA,
        ],
    )
    def agg(h_hbm, src_hbm, dst_hbm, out_hbm, src_v, dst_v, rows_v, acc, sem,
            sem_s):
        c = lax.axis_index("c")
        s = lax.axis_index("s")
        r0 = s * rpt

        @pl.when(s < 15)
        def _():
            pltpu.sync_copy(h_hbm.at[pl.ds(c * seed_stride + r0, rpt)],
                            acc.at[pl.ds(r0, rpt)])

        @pl.when(s == 15)
        def _():
            pltpu.sync_copy(h_hbm.at[pl.ds(c * seed_stride + 15 * rpt, rlast)],
                            acc.at[pl.ds(15 * rpt, rlast)])

        plsc.subcore_barrier()

        def winN(i, carry):
            j0 = i * NBUF
            for b in range(NBUF):
                j = j0 + b
                pltpu.make_async_copy(
                    h_hbm.at[src_v.at[j]], rows_v.at[b], sem).wait()
                pltpu.async_copy(rows_v.at[b], acc.at[dst_v.at[j]], sem_s,
                                 add=True)
                nxt = j + NBUF - 1
                bn_ = (b + NBUF - 1) % NBUF

                @pl.when((nxt < kwh) & (j >= 1))
                def _():
                    pltpu.make_async_copy(
                        rows_v.at[bn_], acc.at[dst_v.at[j]], sem_s).wait()

                @pl.when(nxt < kwh)
                def _():
                    pltpu.async_copy(
                        h_hbm.at[src_v.at[nxt]], rows_v.at[bn_], sem)
            return carry

        for stage in range(nst):
            base = s * kw + stage * kwh
            pltpu.sync_copy(src_hbm.at[c, pl.ds(base, kwh)], src_v)
            pltpu.sync_copy(dst_hbm.at[c, pl.ds(base, kwh)], dst_v)
            for k in range(NBUF - 1):
                pltpu.async_copy(h_hbm.at[src_v.at[k]], rows_v.at[k], sem)
            lax.fori_loop(0, kwh // NBUF, winN, 0)
            for k in range(NBUF):
                pltpu.make_async_copy(
                    rows_v.at[k], acc.at[dst_v.at[0]], sem_s).wait()

        plsc.subcore_barrier()

        @pl.when(s < 15)
        def _():
            pltpu.sync_copy(acc.at[pl.ds(r0, rpt)],
                            out_hbm.at[c, pl.ds(r0, rpt)])

        @pl.when(s == 15)
        def _():
            pltpu.sync_copy(acc.at[pl.ds(15 * rpt, rlast)],
                            out_hbm.at[c, pl.ds(15 * rpt, rlast)])

    return agg(h2, srcb, dstb)


def _tc_mlp1(p01, x, batch3, w1, b1, w2, b2):
    n, bn = N_NODES, 1000

    def body(agg_r, x_r, bat_r, w1_r, b1_r, w2_r, b2_r, h1_r, p1_r):
        a = agg_r[0] + agg_r[1] - x_r[...]
        t = jnp.dot(a, w1_r[...], preferred_element_type=jnp.float32) + b1_r[...]
        t = jnp.maximum(t, 0.0)
        h1 = jnp.dot(t, w2_r[...], preferred_element_type=jnp.float32) + b2_r[...]
        h1 = jnp.maximum(h1, 0.0)
        h1_r[0] = h1[:, :128]
        h1_r[1] = h1[:, 128:]
        gids = lax.broadcasted_iota(jnp.int32, (64, bn), 0)
        oh = (bat_r[0] == gids).astype(jnp.float32)
        part = jnp.dot(oh, h1, preferred_element_type=jnp.float32)

        @pl.when(pl.program_id(0) == 0)
        def _():
            p1_r[...] = jnp.zeros_like(p1_r)

        p1_r[...] += part

    return pl.pallas_call(
        body,
        grid=(n // bn,),
        in_specs=[
            pl.BlockSpec((2, bn, 128), lambda i: (0, i, 0)),
            pl.BlockSpec((bn, 128), lambda i: (i, 0)),
            pl.BlockSpec((1, 1, bn), lambda i: (i, 0, 0)),
            pl.BlockSpec((128, 256), lambda i: (0, 0)),
            pl.BlockSpec((1, 256), lambda i: (0, 0)),
            pl.BlockSpec((256, 256), lambda i: (0, 0)),
            pl.BlockSpec((1, 256), lambda i: (0, 0)),
        ],
        out_specs=[
            pl.BlockSpec((2, bn, 128), lambda i: (0, i, 0)),
            pl.BlockSpec((64, 256), lambda i: (0, 0)),
        ],
        out_shape=[
            jax.ShapeDtypeStruct((2, n, 128), jnp.float32),
            jax.ShapeDtypeStruct((64, 256), jnp.float32),
        ],
    )(p01, x, batch3, w1, b1, w2, b2)


def _tc_mlp2(agg2, batch3, w3, b3, w4, b4):
    n, bn = N_NODES, 1000

    def body(agg_r, bat_r, w3_r, b3_r, w4_r, b4_r, p2_r):
        a = jnp.concatenate([agg_r[0], agg_r[1]], axis=1)
        t = jnp.dot(a, w3_r[...], preferred_element_type=jnp.float32) + b3_r[...]
        t = jnp.maximum(t, 0.0)
        h2 = jnp.dot(t, w4_r[...], preferred_element_type=jnp.float32) + b4_r[...]
        h2 = jnp.maximum(h2, 0.0)
        gids = lax.broadcasted_iota(jnp.int32, (64, bn), 0)
        oh = (bat_r[0] == gids).astype(jnp.float32)
        part = jnp.dot(oh, h2, preferred_element_type=jnp.float32)

        @pl.when(pl.program_id(0) == 0)
        def _():
            p2_r[...] = jnp.zeros_like(p2_r)

        p2_r[...] += part

    return pl.pallas_call(
        body,
        grid=(n // bn,),
        in_specs=[
            pl.BlockSpec((2, bn, 128), lambda i: (0, i, 0)),
            pl.BlockSpec((1, 1, bn), lambda i: (i, 0, 0)),
            pl.BlockSpec((256, 256), lambda i: (0, 0)),
            pl.BlockSpec((1, 256), lambda i: (0, 0)),
            pl.BlockSpec((256, 256), lambda i: (0, 0)),
            pl.BlockSpec((1, 256), lambda i: (0, 0)),
        ],
        out_specs=pl.BlockSpec((64, 256), lambda i: (0, 0)),
        out_shape=jax.ShapeDtypeStruct((64, 256), jnp.float32),
    )(agg2, batch3, w3, b3, w4, b4)


def _tc_head(p1, p2, wl1, bl1, wl2, bl2):
    def body(p1_r, p2_r, w1_r, b1_r, w2_r, b2_r, o_r):
        z = (jnp.dot(p1_r[...], w1_r[0], preferred_element_type=jnp.float32)
             + jnp.dot(p2_r[...], w1_r[1], preferred_element_type=jnp.float32)
             + b1_r[...])
        z = jnp.maximum(z, 0.0)
        o_r[...] = jnp.dot(z, w2_r[...], preferred_element_type=jnp.float32) + b2_r[...]

    return pl.pallas_call(
        body,
        out_shape=jax.ShapeDtypeStruct((64, 10), jnp.float32),
    )(p1, p2, wl1, bl1, wl2, bl2)


def kernel(x, edge_index, batch, W1, b1, g1, be1, W2, b2, W3, b3, g2, be2,
           W4, b4, Wl1, bl1, Wl2, bl2):
    n = x.shape[0]
    e = edge_index.shape[1]
    src = edge_index[0]
    dst = edge_index[1]

    eh = e // 2
    kw1 = -(-(-(-eh // (N_TILES * WIN))) // 8) * 8
    npad1 = N_TILES * kw1 * WIN - eh
    pad1 = jnp.arange(npad1, dtype=jnp.int32) % PAD_ROWS
    srcb1 = jnp.stack([
        jnp.concatenate([src[:eh], pad1]),
        jnp.concatenate([src[eh:], pad1]),
    ]).reshape(2, N_TILES * kw1, WIN)
    dstb1 = jnp.stack([
        jnp.concatenate([dst[:eh], n + pad1]),
        jnp.concatenate([dst[eh:], n + pad1]),
    ]).reshape(2, N_TILES * kw1, WIN)

    kw2 = -(-(-(-e // (N_TILES * WIN))) // 8) * 8
    npad2 = N_TILES * kw2 * WIN - e
    pad2 = jnp.arange(npad2, dtype=jnp.int32) % PAD_ROWS
    srcp2 = jnp.concatenate([src, pad2])
    dstp2 = jnp.concatenate([dst, n + pad2])
    srcb2 = jnp.stack([srcp2, srcp2 + n]).reshape(2, N_TILES * kw2, WIN)
    dstb2 = jnp.stack([dstp2, dstp2]).reshape(2, N_TILES * kw2, WIN)

    sinv = 1.0 / jnp.sqrt(1.0 + 1e-5)
    w1s = g1 * sinv
    W1f = W1 * w1s[None, :]
    b1f = b1 * w1s + be1
    w4s = g2 * sinv
    W4f = W4 * w4s[:, None]
    b4f = b4 + be2 @ W4

    batch3 = batch.reshape(10, 1, 1000)

    agg1 = _sc_agg(x, srcb1, dstb1, 128, kw1, 0)
    h1_both, p1 = _tc_mlp1(agg1, x, batch3, W1f, b1f.reshape(1, -1), W2,
                           b2.reshape(1, -1))

    agg2 = _sc_agg(h1_both.reshape(2 * n, 128), srcb2, dstb2, 128, kw2, n)
    p2 = _tc_mlp2(agg2, batch3, W3, b3.reshape(1, -1), W4f,
                  b4f.reshape(1, -1))

    return _tc_head(p1, p2, Wl1.reshape(2, 256, 128), bl1.reshape(1, -1),
                    Wl2, bl2.reshape(1, -1))


# speedup vs baseline: 1.0183x; 1.0006x over previous
import functools

import jax
import jax.numpy as jnp
from jax import lax
from jax.experimental import pallas as pl
from jax.experimental.pallas import tpu as pltpu
from jax.experimental.pallas import tpu_sc as plsc

N_NODES = 10000
N_TILES = 16
WIN = 64
NBUF = 4
KWH = 40
PAD_ROWS = 8
FH = 128


def _sc_agg(h2, srcb, dstb, kw, wbase, coff_stride):
    n = N_NODES
    rpt = 632
    rlast = n - 15 * rpt
    mesh = plsc.VectorSubcoreMesh(core_axis_name="c", subcore_axis_name="s")
    nst = kw // KWH

    @functools.partial(
        pl.kernel,
        mesh=mesh,
        out_type=jax.ShapeDtypeStruct((2, n, FH), jnp.float32),
        scratch_types=[
            pltpu.VMEM((KWH, WIN), jnp.int32),
            pltpu.VMEM((KWH, WIN), jnp.int32),
            pltpu.VMEM((NBUF, WIN, FH), jnp.float32),
            pltpu.VMEM_SHARED((n + PAD_ROWS, FH), jnp.float32),
            pltpu.SemaphoreType.DMA,
            pltpu.SemaphoreType.DMA,
        ],
    )
    def agg(h_hbm, src_hbm, dst_hbm, out_hbm, src_v, dst_v, rows_v, acc, sem,
            sem_s):
        c = lax.axis_index("c")
        s = lax.axis_index("s")
        r0 = s * rpt
        coff = c * coff_stride

        @pl.when(s < 15)
        def _():
            pltpu.sync_copy(h_hbm.at[pl.ds(r0, rpt), pl.ds(coff, FH)],
                            acc.at[pl.ds(r0, rpt)])

        @pl.when(s == 15)
        def _():
            pltpu.sync_copy(h_hbm.at[pl.ds(15 * rpt, rlast), pl.ds(coff, FH)],
                            acc.at[pl.ds(15 * rpt, rlast)])

        plsc.subcore_barrier()

        def winN(i, carry):
            j0 = i * NBUF
            for b in range(NBUF):
                j = j0 + b
                pltpu.make_async_copy(
                    h_hbm.at[src_v.at[j], pl.ds(coff, FH)],
                    rows_v.at[b], sem).wait()
                pltpu.async_copy(rows_v.at[b], acc.at[dst_v.at[j]], sem_s,
                                 add=True)
                nxt = j + NBUF - 1
                bn_ = (b + NBUF - 1) % NBUF

                @pl.when((nxt < KWH) & (j >= 1))
                def _():
                    pltpu.make_async_copy(
                        rows_v.at[bn_], acc.at[dst_v.at[j]], sem_s).wait()

                @pl.when(nxt < KWH)
                def _():
                    pltpu.async_copy(
                        h_hbm.at[src_v.at[nxt], pl.ds(coff, FH)],
                        rows_v.at[bn_], sem)
            return carry

        def stage_body(st, carry):
            base = pl.multiple_of(c * wbase + s * kw + st * KWH, 8)
            pltpu.sync_copy(src_hbm.at[pl.ds(base, KWH)], src_v)
            pltpu.sync_copy(dst_hbm.at[pl.ds(base, KWH)], dst_v)
            for k in range(NBUF - 1):
                pltpu.async_copy(h_hbm.at[src_v.at[k], pl.ds(coff, FH)],
                                 rows_v.at[k], sem)
            lax.fori_loop(0, KWH // NBUF, winN, 0)
            for k in range(NBUF):
                pltpu.make_async_copy(
                    rows_v.at[k], acc.at[dst_v.at[0]], sem_s).wait()
            return carry

        lax.fori_loop(0, nst, stage_body, 0)

        plsc.subcore_barrier()

        @pl.when(s < 15)
        def _():
            pltpu.sync_copy(acc.at[pl.ds(r0, rpt)],
                            out_hbm.at[c, pl.ds(r0, rpt)])

        @pl.when(s == 15)
        def _():
            pltpu.sync_copy(acc.at[pl.ds(15 * rpt, rlast)],
                            out_hbm.at[c, pl.ds(15 * rpt, rlast)])

    return agg(h2, srcb, dstb)


def _tc_mlp1(p01, x, batch3, w1, b1, w2, b2):
    n, bn = N_NODES, 1000

    def body(agg_r, x_r, bat_r, w1_r, b1_r, w2_r, b2_r, h1_r, p1_r):
        a = agg_r[0] + agg_r[1] - x_r[...]
        t = jnp.dot(a, w1_r[...], preferred_element_type=jnp.float32) + b1_r[...]
        t = jnp.maximum(t, 0.0)
        h1 = jnp.dot(t, w2_r[...], preferred_element_type=jnp.float32) + b2_r[...]
        h1 = jnp.maximum(h1, 0.0)
        h1_r[...] = h1
        gids = lax.broadcasted_iota(jnp.int32, (64, bn), 0)
        oh = (bat_r[0] == gids).astype(jnp.float32)
        part = jnp.dot(oh, h1, preferred_element_type=jnp.float32)

        @pl.when(pl.program_id(0) == 0)
        def _():
            p1_r[...] = jnp.zeros_like(p1_r)

        p1_r[...] += part

    return pl.pallas_call(
        body,
        grid=(n // bn,),
        in_specs=[
            pl.BlockSpec((2, bn, 128), lambda i: (0, i, 0)),
            pl.BlockSpec((bn, 128), lambda i: (i, 0)),
            pl.BlockSpec((1, 1, bn), lambda i: (i, 0, 0)),
            pl.BlockSpec((128, 256), lambda i: (0, 0)),
            pl.BlockSpec((1, 256), lambda i: (0, 0)),
            pl.BlockSpec((256, 256), lambda i: (0, 0)),
            pl.BlockSpec((1, 256), lambda i: (0, 0)),
        ],
        out_specs=[
            pl.BlockSpec((bn, 256), lambda i: (i, 0)),
            pl.BlockSpec((64, 256), lambda i: (0, 0)),
        ],
        out_shape=[
            jax.ShapeDtypeStruct((n, 256), jnp.float32),
            jax.ShapeDtypeStruct((64, 256), jnp.float32),
        ],
    )(p01, x, batch3, w1, b1, w2, b2)


def _tc_mlp2(agg2, batch3, w3, b3, w4, b4, p1, wl1, bl1, wl2, bl2):
    n, bn = N_NODES, 1000
    steps = n // bn

    def body(agg_r, bat_r, w3_r, b3_r, w4_r, b4_r, p1_r, wl1_r, bl1_r,
             wl2_r, bl2_r, z_r, p2_s):
        a = jnp.concatenate([agg_r[0], agg_r[1]], axis=1)
        t = jnp.dot(a, w3_r[...], preferred_element_type=jnp.float32) + b3_r[...]
        t = jnp.maximum(t, 0.0)
        h2 = jnp.dot(t, w4_r[...], preferred_element_type=jnp.float32) + b4_r[...]
        h2 = jnp.maximum(h2, 0.0)
        gids = lax.broadcasted_iota(jnp.int32, (64, bn), 0)
        oh = (bat_r[0] == gids).astype(jnp.float32)
        part = jnp.dot(oh, h2, preferred_element_type=jnp.float32)

        @pl.when(pl.program_id(0) == 0)
        def _():
            p2_s[...] = jnp.zeros_like(p2_s)

        p2_s[...] += part

        @pl.when(pl.program_id(0) == steps - 1)
        def _():
            z = (jnp.dot(p1_r[...], wl1_r[0],
                         preferred_element_type=jnp.float32)
                 + jnp.dot(p2_s[...], wl1_r[1],
                           preferred_element_type=jnp.float32)
                 + bl1_r[...])
            z = jnp.maximum(z, 0.0)
            z_r[...] = jnp.dot(z, wl2_r[...],
                               preferred_element_type=jnp.float32) + bl2_r[...]

    return pl.pallas_call(
        body,
        grid=(steps,),
        in_specs=[
            pl.BlockSpec((2, bn, 128), lambda i: (0, i, 0)),
            pl.BlockSpec((1, 1, bn), lambda i: (i, 0, 0)),
            pl.BlockSpec((256, 256), lambda i: (0, 0)),
            pl.BlockSpec((1, 256), lambda i: (0, 0)),
            pl.BlockSpec((256, 256), lambda i: (0, 0)),
            pl.BlockSpec((1, 256), lambda i: (0, 0)),
            pl.BlockSpec((64, 256), lambda i: (0, 0)),
            pl.BlockSpec((2, 256, 128), lambda i: (0, 0, 0)),
            pl.BlockSpec((1, 128), lambda i: (0, 0)),
            pl.BlockSpec((128, 10), lambda i: (0, 0)),
            pl.BlockSpec((1, 10), lambda i: (0, 0)),
        ],
        out_specs=pl.BlockSpec((64, 10), lambda i: (0, 0)),
        out_shape=jax.ShapeDtypeStruct((64, 10), jnp.float32),
        scratch_shapes=[pltpu.VMEM((64, 256), jnp.float32)],
    )(agg2, batch3, w3, b3, w4, b4, p1, wl1, bl1, wl2, bl2)


def kernel(x, edge_index, batch, W1, b1, g1, be1, W2, b2, W3, b3, g2, be2,
           W4, b4, Wl1, bl1, Wl2, bl2):
    n = x.shape[0]
    e = edge_index.shape[1]
    src = edge_index[0]
    dst = edge_index[1]

    kw1 = -(-(-(-(e // 2) // (N_TILES * WIN))) // KWH) * KWH
    kw2 = 2 * kw1
    npad = 2 * N_TILES * kw1 * WIN - e
    pad_ids = jnp.arange(npad, dtype=jnp.int32) % PAD_ROWS
    srcv = jnp.concatenate([src, pad_ids]).reshape(2 * N_TILES * kw1, WIN)
    dstv = jnp.concatenate([dst, n + pad_ids]).reshape(2 * N_TILES * kw1, WIN)

    sinv = 1.0 / jnp.sqrt(1.0 + 1e-5)
    w1s = g1 * sinv
    W1f = W1 * w1s[None, :]
    b1f = b1 * w1s + be1
    w4s = g2 * sinv
    W4f = W4 * w4s[:, None]
    b4f = b4 + be2 @ W4

    batch3 = batch.reshape(10, 1, 1000)

    agg1 = _sc_agg(x, srcv, dstv, kw1, N_TILES * kw1, 0)
    h1, p1 = _tc_mlp1(agg1, x, batch3, W1f, b1f.reshape(1, -1), W2,
                      b2.reshape(1, -1))

    agg2 = _sc_agg(h1, srcv, dstv, kw2, 0, 128)
    return _tc_mlp2(agg2, batch3, W3, b3.reshape(1, -1), W4f,
                    b4f.reshape(1, -1), p1, Wl1.reshape(2, 256, 128),
                    bl1.reshape(1, -1), Wl2, bl2.reshape(1, -1))


# speedup vs baseline: 1.2289x; 1.2069x over previous
import functools

import jax
import jax.numpy as jnp
from jax import lax
from jax.experimental import pallas as pl
from jax.experimental.pallas import tpu as pltpu
from jax.experimental.pallas import tpu_sc as plsc

N_NODES = 10000
N_TILES = 16
WIN = 64
NBUF = 4
KWH = 40
PAD_ROWS = 128
FH = 128


def _sc_agg(h2, srcb, dstb, kw, wbase, coff_stride):
    n = N_NODES
    rpt = 632
    rlast = n - 15 * rpt
    mesh = plsc.VectorSubcoreMesh(core_axis_name="c", subcore_axis_name="s")
    nst = kw // KWH

    @functools.partial(
        pl.kernel,
        mesh=mesh,
        out_type=jax.ShapeDtypeStruct((2, n, FH), jnp.float32),
        scratch_types=[
            pltpu.VMEM((KWH, WIN), jnp.int32),
            pltpu.VMEM((KWH, WIN), jnp.int32),
            pltpu.VMEM((NBUF, WIN, FH), jnp.float32),
            pltpu.VMEM_SHARED((n + PAD_ROWS, FH), jnp.float32),
            pltpu.SemaphoreType.DMA,
            pltpu.SemaphoreType.DMA,
        ],
    )
    def agg(h_hbm, src_hbm, dst_hbm, out_hbm, src_v, dst_v, rows_v, acc, sem,
            sem_s):
        c = lax.axis_index("c")
        s = lax.axis_index("s")
        r0 = s * rpt
        coff = c * coff_stride

        @pl.when(s < 15)
        def _():
            pltpu.sync_copy(h_hbm.at[pl.ds(r0, rpt), pl.ds(coff, FH)],
                            acc.at[pl.ds(r0, rpt)])

        @pl.when(s == 15)
        def _():
            pltpu.sync_copy(h_hbm.at[pl.ds(15 * rpt, rlast), pl.ds(coff, FH)],
                            acc.at[pl.ds(15 * rpt, rlast)])

        plsc.subcore_barrier()

        def winN(i, carry):
            j0 = i * NBUF
            for b in range(NBUF):
                j = j0 + b
                pltpu.make_async_copy(
                    h_hbm.at[src_v.at[j], pl.ds(coff, FH)],
                    rows_v.at[b], sem).wait()
                pltpu.async_copy(rows_v.at[b], acc.at[dst_v.at[j]], sem_s,
                                 add=True)
                nxt = j + NBUF - 1
                bn_ = (b + NBUF - 1) % NBUF

                @pl.when((nxt < KWH) & (j >= 1))
                def _():
                    pltpu.make_async_copy(
                        rows_v.at[bn_], acc.at[dst_v.at[j]], sem_s).wait()

                @pl.when(nxt < KWH)
                def _():
                    pltpu.async_copy(
                        h_hbm.at[src_v.at[nxt], pl.ds(coff, FH)],
                        rows_v.at[bn_], sem)
            return carry

        def stage_body(st, carry):
            base = pl.multiple_of(c * wbase + s * kw + st * KWH, 8)
            pltpu.sync_copy(src_hbm.at[pl.ds(base, KWH)], src_v)
            pltpu.sync_copy(dst_hbm.at[pl.ds(base, KWH)], dst_v)
            for k in range(NBUF - 1):
                pltpu.async_copy(h_hbm.at[src_v.at[k], pl.ds(coff, FH)],
                                 rows_v.at[k], sem)
            lax.fori_loop(0, KWH // NBUF, winN, 0)
            for k in range(NBUF):
                pltpu.make_async_copy(
                    rows_v.at[k], acc.at[dst_v.at[0]], sem_s).wait()
            return carry

        lax.fori_loop(0, nst, stage_body, 0)

        plsc.subcore_barrier()

        @pl.when(s < 15)
        def _():
            pltpu.sync_copy(acc.at[pl.ds(r0, rpt)],
                            out_hbm.at[c, pl.ds(r0, rpt)])

        @pl.when(s == 15)
        def _():
            pltpu.sync_copy(acc.at[pl.ds(15 * rpt, rlast)],
                            out_hbm.at[c, pl.ds(15 * rpt, rlast)])

    return agg(h2, srcb, dstb)


def _tc_mlp1(p01, x, batch3, w1, b1, w2, b2):
    n, bn = N_NODES, 1000

    def body(agg_r, x_r, bat_r, w1_r, b1_r, w2_r, b2_r, h1_r, p1_r):
        a = agg_r[0] + agg_r[1] - x_r[...]
        t = jnp.dot(a, w1_r[...], preferred_element_type=jnp.float32) + b1_r[...]
        t = jnp.maximum(t, 0.0)
        h1 = jnp.dot(t, w2_r[...], preferred_element_type=jnp.float32) + b2_r[...]
        h1 = jnp.maximum(h1, 0.0)
        h1_r[...] = h1
        gids = lax.broadcasted_iota(jnp.int32, (64, bn), 0)
        oh = (bat_r[0] == gids).astype(jnp.float32)
        part = jnp.dot(oh, h1, preferred_element_type=jnp.float32)

        @pl.when(pl.program_id(0) == 0)
        def _():
            p1_r[...] = jnp.zeros_like(p1_r)

        p1_r[...] += part

    return pl.pallas_call(
        body,
        grid=(n // bn,),
        in_specs=[
            pl.BlockSpec((2, bn, 128), lambda i: (0, i, 0)),
            pl.BlockSpec((bn, 128), lambda i: (i, 0)),
            pl.BlockSpec((1, 1, bn), lambda i: (i, 0, 0)),
            pl.BlockSpec((128, 256), lambda i: (0, 0)),
            pl.BlockSpec((1, 256), lambda i: (0, 0)),
            pl.BlockSpec((256, 256), lambda i: (0, 0)),
            pl.BlockSpec((1, 256), lambda i: (0, 0)),
        ],
        out_specs=[
            pl.BlockSpec((bn, 256), lambda i: (i, 0)),
            pl.BlockSpec((64, 256), lambda i: (0, 0)),
        ],
        out_shape=[
            jax.ShapeDtypeStruct((n, 256), jnp.float32),
            jax.ShapeDtypeStruct((64, 256), jnp.float32),
        ],
    )(p01, x, batch3, w1, b1, w2, b2)


def _tc_mlp2(agg2, batch3, w3, b3, w4, b4, p1, wl1, bl1, wl2, bl2):
    n, bn = N_NODES, 1000
    steps = n // bn

    def body(agg_r, bat_r, w3_r, b3_r, w4_r, b4_r, p1_r, wl1_r, bl1_r,
             wl2_r, bl2_r, z_r, p2_s):
        a = jnp.concatenate([agg_r[0], agg_r[1]], axis=1)
        t = jnp.dot(a, w3_r[...], preferred_element_type=jnp.float32) + b3_r[...]
        t = jnp.maximum(t, 0.0)
        h2 = jnp.dot(t, w4_r[...], preferred_element_type=jnp.float32) + b4_r[...]
        h2 = jnp.maximum(h2, 0.0)
        gids = lax.broadcasted_iota(jnp.int32, (64, bn), 0)
        oh = (bat_r[0] == gids).astype(jnp.float32)
        part = jnp.dot(oh, h2, preferred_element_type=jnp.float32)

        @pl.when(pl.program_id(0) == 0)
        def _():
            p2_s[...] = jnp.zeros_like(p2_s)

        p2_s[...] += part

        @pl.when(pl.program_id(0) == steps - 1)
        def _():
            z = (jnp.dot(p1_r[...], wl1_r[0],
                         preferred_element_type=jnp.float32)
                 + jnp.dot(p2_s[...], wl1_r[1],
                           preferred_element_type=jnp.float32)
                 + bl1_r[...])
            z = jnp.maximum(z, 0.0)
            z_r[...] = jnp.dot(z, wl2_r[...],
                               preferred_element_type=jnp.float32) + bl2_r[...]

    return pl.pallas_call(
        body,
        grid=(steps,),
        in_specs=[
            pl.BlockSpec((2, bn, 128), lambda i: (0, i, 0)),
            pl.BlockSpec((1, 1, bn), lambda i: (i, 0, 0)),
            pl.BlockSpec((256, 256), lambda i: (0, 0)),
            pl.BlockSpec((1, 256), lambda i: (0, 0)),
            pl.BlockSpec((256, 256), lambda i: (0, 0)),
            pl.BlockSpec((1, 256), lambda i: (0, 0)),
            pl.BlockSpec((64, 256), lambda i: (0, 0)),
            pl.BlockSpec((2, 256, 128), lambda i: (0, 0, 0)),
            pl.BlockSpec((1, 128), lambda i: (0, 0)),
            pl.BlockSpec((128, 10), lambda i: (0, 0)),
            pl.BlockSpec((1, 10), lambda i: (0, 0)),
        ],
        out_specs=pl.BlockSpec((64, 10), lambda i: (0, 0)),
        out_shape=jax.ShapeDtypeStruct((64, 10), jnp.float32),
        scratch_shapes=[pltpu.VMEM((64, 256), jnp.float32)],
    )(agg2, batch3, w3, b3, w4, b4, p1, wl1, bl1, wl2, bl2)


def kernel(x, edge_index, batch, W1, b1, g1, be1, W2, b2, W3, b3, g2, be2,
           W4, b4, Wl1, bl1, Wl2, bl2):
    n = x.shape[0]
    e = edge_index.shape[1]
    src = edge_index[0]
    dst = edge_index[1]

    kw1 = -(-(-(-(e // 2) // (N_TILES * WIN))) // KWH) * KWH
    kw2 = 2 * kw1
    npad = 2 * N_TILES * kw1 * WIN - e
    pad_ids = jnp.arange(npad, dtype=jnp.int32) % PAD_ROWS
    srcv = jnp.concatenate([src, pad_ids]).reshape(2 * N_TILES * kw1, WIN)
    dstv = jnp.concatenate([dst, n + pad_ids]).reshape(2 * N_TILES * kw1, WIN)

    sinv = 1.0 / jnp.sqrt(1.0 + 1e-5)
    w1s = g1 * sinv
    W1f = W1 * w1s[None, :]
    b1f = b1 * w1s + be1
    w4s = g2 * sinv
    W4f = W4 * w4s[:, None]
    b4f = b4 + be2 @ W4

    batch3 = batch.reshape(10, 1, 1000)

    agg1 = _sc_agg(x, srcv, dstv, kw1, N_TILES * kw1, 0)
    h1, p1 = _tc_mlp1(agg1, x, batch3, W1f, b1f.reshape(1, -1), W2,
                      b2.reshape(1, -1))

    agg2 = _sc_agg(h1, srcv, dstv, kw2, 0, 128)
    return _tc_mlp2(agg2, batch3, W3, b3.reshape(1, -1), W4f,
                    b4f.reshape(1, -1), p1, Wl1.reshape(2, 256, 128),
                    bl1.reshape(1, -1), Wl2, bl2.reshape(1, -1))
